# baseline (speedup 1.0000x reference)
import functools

import jax
import jax.numpy as jnp
from jax import lax
from jax.experimental import pallas as pl
from jax.experimental.pallas import tpu as pltpu
from jax.experimental.pallas import tpu_sc as plsc

B = 16384
R = 8
V = 64
NDIMS = 7
NC, NS, L = 2, 16, 16
NW = NC * NS
BPW = B // NW
GROUPS = BPW // L
SE = R + 1
SM = R + 1


def _tt_body(idx_hbm, t0_hbm, t1_hbm, t2_hbm, t3_hbm, t4_hbm, t5_hbm,
             t6_hbm, out_hbm, idx_v, t0_v, t1_v, t2_v, t3_v, t4_v, t5_v,
             t6_v, out_v, sem):
    wid = lax.axis_index("s") * NC + lax.axis_index("c")
    base = wid * BPW

    copies = [
        pltpu.async_copy(t0_hbm, t0_v, sem),
        pltpu.async_copy(t1_hbm, t1_v, sem),
        pltpu.async_copy(t2_hbm, t2_v, sem),
        pltpu.async_copy(t3_hbm, t3_v, sem),
        pltpu.async_copy(t4_hbm, t4_v, sem),
        pltpu.async_copy(t5_hbm, t5_v, sem),
        pltpu.async_copy(t6_hbm, t6_v, sem),
        pltpu.async_copy(idx_hbm.at[pl.ds(wid * (NDIMS * BPW), NDIMS * BPW)],
                         idx_v, sem),
    ]
    for c in copies:
        c.wait()

    tmid = [t1_v, t2_v, t3_v, t4_v, t5_v]

    @plsc.parallel_loop(0, GROUPS)
    def _group(g):
        o = g * L
        i0 = idx_v[pl.ds(0 * BPW + o, L)] * SE
        res = [plsc.load_gather(t0_v, [i0 + j]) for j in range(R)]
        for k in range(1, 6):
            ik = idx_v[pl.ds(k * BPW + o, L)] * SM
            tk = tmid[k - 1]
            new = []
            for l in range(R):
                acc = res[0] * plsc.load_gather(tk, [ik + l])
                for j in range(1, R):
                    acc = acc + res[j] * plsc.load_gather(
                        tk, [ik + (j * (V * SM) + l)])
                new.append(acc)
            res = new
        i6 = idx_v[pl.ds(6 * BPW + o, L)]
        q = res[0] * plsc.load_gather(t6_v, [i6])
        for j in range(1, R):
            q = q + res[j] * plsc.load_gather(t6_v, [i6 + j * V])
        out_v[pl.ds(o, L)] = q

    pltpu.sync_copy(out_v, out_hbm.at[pl.ds(base, BPW)])


_tt_gather = functools.partial(
    pl.kernel,
    out_type=jax.ShapeDtypeStruct((B,), jnp.float32),
    mesh=plsc.VectorSubcoreMesh(core_axis_name="c", subcore_axis_name="s",
                                num_cores=NC, num_subcores=NS),
    compiler_params=pltpu.CompilerParams(needs_layout_passes=False),
    scratch_types=[
        pltpu.VMEM((NDIMS * BPW,), jnp.int32),
        pltpu.VMEM((V * SE,), jnp.float32),
        pltpu.VMEM((R * V * SM,), jnp.float32),
        pltpu.VMEM((R * V * SM,), jnp.float32),
        pltpu.VMEM((R * V * SM,), jnp.float32),
        pltpu.VMEM((R * V * SM,), jnp.float32),
        pltpu.VMEM((R * V * SM,), jnp.float32),
        pltpu.VMEM((R * V,), jnp.float32),
        pltpu.VMEM((BPW,), jnp.float32),
        pltpu.SemaphoreType.DMA,
    ],
)(_tt_body)


def kernel(states, actions, core0, core1, core2, core3, core4, core5, core6):
    idxp = (jnp.concatenate([states.T, actions.T], axis=0)
            .reshape(NDIMS, NW, BPW).transpose(1, 0, 2).reshape(-1))
    t0 = jnp.pad(core0.reshape(V, R), ((0, 0), (0, SE - R))).reshape(-1)
    tmid = [jnp.pad(c, ((0, 0), (0, 0), (0, SM - R))).reshape(-1)
            for c in (core1, core2, core3, core4, core5)]
    t6 = core6.reshape(-1)
    return _tt_gather(idxp, t0, *tmid, t6)
